# baseline (speedup 1.0000x reference)
import functools

import jax
import jax.numpy as jnp
from jax import lax
from jax.experimental import pallas as pl
from jax.experimental.pallas import tpu as pltpu
from jax.experimental.pallas import tpu_sc as plsc

N_NODES = 10000
N_EDGES = 320000
N_GRAPHS = 64

_NC = 2
_NS = 16
_NW = _NC * _NS


def _dot_bf16(a, b):
    return jnp.dot(a.astype(jnp.bfloat16), b.astype(jnp.bfloat16),
                   preferred_element_type=jnp.float32)


def _leaky(x):
    return jnp.where(x > 0, x, 0.2 * x)


def _edge_gather(xpad, dst, src):
    n, c = xpad.shape
    e = dst.shape[0]
    epw = e // _NW
    k = 80
    nch = epw // k
    mesh = plsc.VectorSubcoreMesh(core_axis_name="c", subcore_axis_name="s")
    out = jax.ShapeDtypeStruct((e, c), jnp.float32)

    @functools.partial(
        pl.kernel, mesh=mesh,
        out_type=(out, out),
        scratch_types=[
            pltpu.VMEM((k,), jnp.int32),
            pltpu.VMEM((k,), jnp.int32),
            pltpu.VMEM((k, c), jnp.float32),
            pltpu.VMEM((k, c), jnp.float32),
            pltpu.SemaphoreType.DMA,
            pltpu.SemaphoreType.DMA,
        ],
    )
    def kern(x_hbm, dst_hbm, src_hbm, gi_hbm, gj_hbm, dbuf, sbuf, pbuf, qbuf,
             sem1, sem2):
        wid = lax.axis_index("s") * _NC + lax.axis_index("c")

        def chunk(j, carry):
            base = wid * epw + j * k
            pltpu.sync_copy(dst_hbm.at[pl.ds(base, k)], dbuf)
            pltpu.sync_copy(src_hbm.at[pl.ds(base, k)], sbuf)
            cp1 = pltpu.async_copy(x_hbm.at[dbuf], pbuf, sem1)
            cp2 = pltpu.async_copy(x_hbm.at[sbuf], qbuf, sem2)
            cp1.wait()
            cp2.wait()
            pltpu.sync_copy(pbuf, gi_hbm.at[pl.ds(base, k)])
            pltpu.sync_copy(qbuf, gj_hbm.at[pl.ds(base, k)])
            return carry

        lax.fori_loop(0, nch, chunk, 0)

    return kern(xpad, dst, src)


def _edge_mlp(gi, gj, wa, sa, ba, wb, sb, bb):
    e, cpad = gi.shape
    cout = wa.shape[1]
    blk = 1280
    steps = e // blk

    def body(gi_ref, gj_ref, wa_ref, sa_ref, ba_ref, wb_ref, sb_ref, bb_ref,
             h_ref):
        xi = gi_ref[...]
        dj = gj_ref[...] - xi
        m = jnp.concatenate([xi, dj], axis=1)
        y = _dot_bf16(m, wa_ref[...]) * sa_ref[...] + ba_ref[...]
        h1 = _leaky(y)
        y2 = _dot_bf16(h1, wb_ref[...]) * sb_ref[...] + bb_ref[...]
        h_ref[...] = _leaky(y2)

    return pl.pallas_call(
        body,
        grid=(steps,),
        in_specs=[
            pl.BlockSpec((blk, cpad), lambda i: (i, 0)),
            pl.BlockSpec((blk, cpad), lambda i: (i, 0)),
            pl.BlockSpec((2 * cpad, cout), lambda i: (0, 0)),
            pl.BlockSpec((1, cout), lambda i: (0, 0)),
            pl.BlockSpec((1, cout), lambda i: (0, 0)),
            pl.BlockSpec((cout, cout), lambda i: (0, 0)),
            pl.BlockSpec((1, cout), lambda i: (0, 0)),
            pl.BlockSpec((1, cout), lambda i: (0, 0)),
        ],
        out_specs=pl.BlockSpec((blk, cout), lambda i: (i, 0)),
        out_shape=jax.ShapeDtypeStruct((e, cout), jnp.float32),
    )(gi, gj, wa, sa, ba, wb, sb, bb)


def _skip_mm(xin, ws, ss, bs):
    n, cin = xin.shape
    c = ws.shape[1]
    blk = 2000

    def body(x_ref, w_ref, s_ref, b_ref, sk_ref):
        sk_ref[...] = _dot_bf16(x_ref[...], w_ref[...]) * s_ref[...] + b_ref[...]

    return pl.pallas_call(
        body,
        grid=(n // blk,),
        in_specs=[
            pl.BlockSpec((blk, cin), lambda i: (i, 0)),
            pl.BlockSpec((cin, c), lambda i: (0, 0)),
            pl.BlockSpec((1, c), lambda i: (0, 0)),
            pl.BlockSpec((1, c), lambda i: (0, 0)),
        ],
        out_specs=pl.BlockSpec((blk, c), lambda i: (i, 0)),
        out_shape=jax.ShapeDtypeStruct((n, c), jnp.float32),
    )(xin, ws, ss, bs)


def _scatter_max(h, dst3, sk):
    e, c = h.shape
    n = sk.shape[0]
    ch = 512
    steps = e // ch

    def body(h_ref, d_ref, sk_ref, out_ref, agg_ref):
        i = pl.program_id(0)

        @pl.when(i == 0)
        def _():
            agg_ref[...] = jnp.full((n, c), -jnp.inf, jnp.float32)

        def upd(ee, cc):
            dd = d_ref[0, 0, ee]
            agg_ref[pl.ds(dd, 1), :] = jnp.maximum(agg_ref[pl.ds(dd, 1), :],
                                                   h_ref[pl.ds(ee, 1), :])
            return cc

        lax.fori_loop(0, ch, upd, 0)

        @pl.when(i == steps - 1)
        def _():
            a = agg_ref[...]
            a = jnp.where(a == -jnp.inf, 0.0, a)
            y = a + sk_ref[...]
            out_ref[...] = _leaky(y)

    return pl.pallas_call(
        body,
        grid=(steps,),
        in_specs=[
            pl.BlockSpec((ch, c), lambda i: (i, 0)),
            pl.BlockSpec((1, 1, ch), lambda i: (i, 0, 0),
                         memory_space=pltpu.SMEM),
            pl.BlockSpec((n, c), lambda i: (0, 0)),
        ],
        out_specs=pl.BlockSpec((n, c), lambda i: (0, 0)),
        out_shape=jax.ShapeDtypeStruct((n, c), jnp.float32),
        scratch_shapes=[pltpu.VMEM((n, c), jnp.float32)],
    )(h, dst3, sk)


def _pool_classify(y, bcols, brows, w1, b1, s1, be1, w2, b2, s2, be2, w3, b3):
    npad, c = y.shape

    def body(y_ref, bc_ref, br_ref, w1_ref, b1_ref, s1_ref, be1_ref, w2_ref,
             b2_ref, s2_ref, be2_ref, w3_ref, b3_ref, out_ref, gmax_ref):
        gids = lax.broadcasted_iota(jnp.int32, (N_GRAPHS, 1), 0)
        onehot = (bc_ref[...] == gids).astype(jnp.float32)
        yv = y_ref[...]
        sums = jnp.dot(onehot, yv, precision=lax.Precision.HIGHEST,
                       preferred_element_type=jnp.float32)
        counts = jnp.sum(onehot, axis=1, keepdims=True)
        gmean = sums / jnp.maximum(counts, 1.0)

        br = br_ref[...]

        def gmax_step(g, cc):
            m = br == g
            ym = jnp.where(m, yv, -jnp.inf)
            gmax_ref[pl.ds(g, 1), :] = jnp.max(ym, axis=0, keepdims=True)
            return cc

        lax.fori_loop(0, N_GRAPHS, gmax_step, 0)
        gmax = gmax_ref[...]
        gmax = jnp.where(gmax == -jnp.inf, 0.0, gmax)

        z = jnp.concatenate([gmean, gmax], axis=1)
        z = _leaky((_dot_bf16(z, w1_ref[...]) + b1_ref[...]) * s1_ref[...]
                   + be1_ref[...])
        z = _leaky((_dot_bf16(z, w2_ref[...]) + b2_ref[...]) * s2_ref[...]
                   + be2_ref[...])
        out_ref[...] = _dot_bf16(z, w3_ref[...]) + b3_ref[...]

    return pl.pallas_call(
        body,
        out_shape=jax.ShapeDtypeStruct((N_GRAPHS, 2), jnp.float32),
        scratch_shapes=[pltpu.VMEM((N_GRAPHS, c), jnp.float32)],
    )(y, bcols, brows, w1, b1, s1, be1, w2, b2, s2, be2, w3, b3)


def _bn_scale(g, eps=1e-5):
    return (g / jnp.sqrt(1.0 + eps))[None, :]


def kernel(x, edge_index, batch, params):
    dst = edge_index[1]
    src = edge_index[0]
    dst3 = dst.reshape(N_EDGES // 512, 1, 512)

    h = x
    for name, cin in (('ec1', 7), ('ec2', 64), ('ec3', 128)):
        p = params[name]
        cpad = max(cin, 128)
        xpad = jnp.pad(h, ((0, 0), (0, cpad - cin))) if cpad != cin else h
        wa = p['Wa']
        wa_pad = jnp.zeros((2 * cpad, wa.shape[1]), jnp.float32)
        wa_pad = wa_pad.at[:cin].set(wa[:cin]).at[cpad:cpad + cin].set(wa[cin:])
        gi, gj = _edge_gather(xpad, dst, src)
        hh = _edge_mlp(gi, gj, wa_pad, _bn_scale(p['ga']), p['ba'][None, :],
                       p['Wb'], _bn_scale(p['gb']), p['bb'][None, :])
        sk = _skip_mm(h, p['Ws'], _bn_scale(p['gs']), p['bs'][None, :])
        h = _scatter_max(hh, dst3, sk)

    npad = 10112
    ypad = jnp.pad(h, ((0, npad - N_NODES), (0, 0)))
    bpad = jnp.pad(batch, (0, npad - N_NODES), constant_values=N_GRAPHS)
    bcols = bpad.reshape(1, npad)
    brows = bpad.reshape(npad, 1)

    c = params['cls']
    return _pool_classify(
        ypad, bcols, brows,
        c['W1'], c['b1'][None, :], _bn_scale(c['g1']), c['be1'][None, :],
        c['W2'], c['b2'][None, :], _bn_scale(c['g2']), c['be2'][None, :],
        c['W3'], c['b3'][None, :])


# speedup vs baseline: 1.3620x; 1.3620x over previous
import functools

import jax
import jax.numpy as jnp
from jax import lax
from jax.experimental import pallas as pl
from jax.experimental.pallas import tpu as pltpu
from jax.experimental.pallas import tpu_sc as plsc

N_NODES = 10000
N_EDGES = 320000
N_GRAPHS = 64

_NC = 2
_NS = 16
_NW = _NC * _NS


def _dot_bf16(a, b):
    return jnp.dot(a.astype(jnp.bfloat16), b.astype(jnp.bfloat16),
                   preferred_element_type=jnp.float32)


def _leaky(x):
    return jnp.where(x > 0, x, 0.2 * x)


def _edge_gather(xpad, dst, src):
    n, c = xpad.shape
    e = dst.shape[0]
    epw = e // _NW
    k = 80
    nch = epw // k
    mesh = plsc.VectorSubcoreMesh(core_axis_name="c", subcore_axis_name="s")
    out = jax.ShapeDtypeStruct((e, c), jnp.float32)

    @functools.partial(
        pl.kernel, mesh=mesh,
        out_type=(out, out),
        scratch_types=[
            pltpu.VMEM((k,), jnp.int32),
            pltpu.VMEM((k,), jnp.int32),
            pltpu.VMEM((k, c), jnp.float32),
            pltpu.VMEM((k, c), jnp.float32),
            pltpu.SemaphoreType.DMA,
            pltpu.SemaphoreType.DMA,
        ],
    )
    def kern(x_hbm, dst_hbm, src_hbm, gi_hbm, gj_hbm, dbuf, sbuf, pbuf, qbuf,
             sem1, sem2):
        wid = lax.axis_index("s") * _NC + lax.axis_index("c")

        def chunk(j, carry):
            base = wid * epw + j * k
            pltpu.sync_copy(dst_hbm.at[pl.ds(base, k)], dbuf)
            pltpu.sync_copy(src_hbm.at[pl.ds(base, k)], sbuf)
            cp1 = pltpu.async_copy(x_hbm.at[dbuf], pbuf, sem1)
            cp2 = pltpu.async_copy(x_hbm.at[sbuf], qbuf, sem2)
            cp1.wait()
            cp2.wait()
            pltpu.sync_copy(pbuf, gi_hbm.at[pl.ds(base, k)])
            pltpu.sync_copy(qbuf, gj_hbm.at[pl.ds(base, k)])
            return carry

        lax.fori_loop(0, nch, chunk, 0)

    return kern(xpad, dst, src)


def _edge_mlp(gi, gj, wa, sa, ba, wb, sb, bb):
    e, cpad = gi.shape
    cout = wa.shape[1]
    cout2 = wb.shape[1]
    blk = 1280
    steps = e // blk

    def body(gi_ref, gj_ref, wa_ref, sa_ref, ba_ref, wb_ref, sb_ref, bb_ref,
             h_ref):
        xi = gi_ref[...]
        dj = gj_ref[...] - xi
        m = jnp.concatenate([xi, dj], axis=1)
        y = _dot_bf16(m, wa_ref[...]) * sa_ref[...] + ba_ref[...]
        h1 = _leaky(y)
        y2 = _dot_bf16(h1, wb_ref[...]) * sb_ref[...] + bb_ref[...]
        h_ref[...] = _leaky(y2)

    return pl.pallas_call(
        body,
        grid=(steps,),
        in_specs=[
            pl.BlockSpec((blk, cpad), lambda i: (i, 0)),
            pl.BlockSpec((blk, cpad), lambda i: (i, 0)),
            pl.BlockSpec((2 * cpad, cout), lambda i: (0, 0)),
            pl.BlockSpec((1, cout), lambda i: (0, 0)),
            pl.BlockSpec((1, cout), lambda i: (0, 0)),
            pl.BlockSpec((cout, cout2), lambda i: (0, 0)),
            pl.BlockSpec((1, cout2), lambda i: (0, 0)),
            pl.BlockSpec((1, cout2), lambda i: (0, 0)),
        ],
        out_specs=pl.BlockSpec((blk, cout2), lambda i: (i, 0)),
        out_shape=jax.ShapeDtypeStruct((e, cout2), jnp.float32),
    )(gi, gj, wa, sa, ba, wb, sb, bb)


def _skip_mm(xin, ws, ss, bs):
    n, cin = xin.shape
    c = ws.shape[1]
    blk = 2000

    def body(x_ref, w_ref, s_ref, b_ref, sk_ref):
        sk_ref[...] = _dot_bf16(x_ref[...], w_ref[...]) * s_ref[...] + b_ref[...]

    return pl.pallas_call(
        body,
        grid=(n // blk,),
        in_specs=[
            pl.BlockSpec((blk, cin), lambda i: (i, 0)),
            pl.BlockSpec((cin, c), lambda i: (0, 0)),
            pl.BlockSpec((1, c), lambda i: (0, 0)),
            pl.BlockSpec((1, c), lambda i: (0, 0)),
        ],
        out_specs=pl.BlockSpec((blk, c), lambda i: (i, 0)),
        out_shape=jax.ShapeDtypeStruct((n, c), jnp.float32),
    )(xin, ws, ss, bs)


def _scatter_max(h, dst3, sk):
    e, c = h.shape
    n = sk.shape[0]
    ch = 512
    steps = e // ch
    nt = 2 if c > 128 else 4

    def body(h_ref, d_ref, sk_ref, out_ref, agg_ref):
        i = pl.program_id(0)

        @pl.when(i == 0)
        def _():
            agg_ref[...] = jnp.full((nt, n, c), -jnp.inf, jnp.float32)

        def upd(g, cc):
            for t in range(nt):
                ee = g * nt + t
                dd = d_ref[0, 0, ee]
                agg_ref[t, pl.ds(dd, 1), :] = jnp.maximum(
                    agg_ref[t, pl.ds(dd, 1), :], h_ref[pl.ds(ee, 1), :])
            return cc

        lax.fori_loop(0, ch // nt, upd, 0)

        @pl.when(i == steps - 1)
        def _():
            a = jnp.max(agg_ref[...], axis=0)
            a = jnp.where(a == -jnp.inf, 0.0, a)
            y = a + sk_ref[...]
            out_ref[...] = _leaky(y)

    return pl.pallas_call(
        body,
        grid=(steps,),
        in_specs=[
            pl.BlockSpec((ch, c), lambda i: (i, 0)),
            pl.BlockSpec((1, 1, ch), lambda i: (i, 0, 0),
                         memory_space=pltpu.SMEM),
            pl.BlockSpec((n, c), lambda i: (0, 0)),
        ],
        out_specs=pl.BlockSpec((n, c), lambda i: (0, 0)),
        out_shape=jax.ShapeDtypeStruct((n, c), jnp.float32),
        scratch_shapes=[pltpu.VMEM((nt, n, c), jnp.float32)],
    )(h, dst3, sk)


def _pool_classify(y, bcols, brows, w1, b1, s1, be1, w2, b2, s2, be2, w3, b3):
    npad, c = y.shape

    def body(y_ref, bc_ref, br_ref, w1_ref, b1_ref, s1_ref, be1_ref, w2_ref,
             b2_ref, s2_ref, be2_ref, w3_ref, b3_ref, out_ref, gmax_ref):
        gids = lax.broadcasted_iota(jnp.int32, (N_GRAPHS, 1), 0)
        onehot = (bc_ref[...] == gids).astype(jnp.float32)
        yv = y_ref[...]
        sums = jnp.dot(onehot, yv, precision=lax.Precision.HIGHEST,
                       preferred_element_type=jnp.float32)
        counts = jnp.sum(onehot, axis=1, keepdims=True)
        gmean = sums / jnp.maximum(counts, 1.0)

        br = br_ref[...]

        def gmax_step(g, cc):
            m = br == g
            ym = jnp.where(m, yv, -jnp.inf)
            gmax_ref[pl.ds(g, 1), :] = jnp.max(ym, axis=0, keepdims=True)
            return cc

        lax.fori_loop(0, N_GRAPHS, gmax_step, 0)
        gmax = gmax_ref[...]
        gmax = jnp.where(gmax == -jnp.inf, 0.0, gmax)

        z = jnp.concatenate([gmean, gmax], axis=1)
        z = _leaky((_dot_bf16(z, w1_ref[...]) + b1_ref[...]) * s1_ref[...]
                   + be1_ref[...])
        z = _leaky((_dot_bf16(z, w2_ref[...]) + b2_ref[...]) * s2_ref[...]
                   + be2_ref[...])
        out_ref[...] = _dot_bf16(z, w3_ref[...]) + b3_ref[...]

    return pl.pallas_call(
        body,
        out_shape=jax.ShapeDtypeStruct((N_GRAPHS, 2), jnp.float32),
        scratch_shapes=[pltpu.VMEM((N_GRAPHS, c), jnp.float32)],
    )(y, bcols, brows, w1, b1, s1, be1, w2, b2, s2, be2, w3, b3)


def _bn_scale(g, eps=1e-5):
    return (g / jnp.sqrt(1.0 + eps))[None, :]


def kernel(x, edge_index, batch, params):
    dst = edge_index[1]
    src = edge_index[0]
    dst3 = dst.reshape(N_EDGES // 512, 1, 512)

    h = x
    for name in ('ec1', 'ec2', 'ec3'):
        p = params[name]
        cin = h.shape[1]
        xg = jnp.pad(h, ((0, 0), (0, 128 - cin))) if cin < 128 else h
        cpin = xg.shape[1]
        wa = p['Wa']
        wa_pad = jnp.zeros((2 * cpin, wa.shape[1]), jnp.float32)
        wa_pad = wa_pad.at[:cin].set(wa[:cin]).at[cpin:cpin + cin].set(wa[cin:])
        gi, gj = _edge_gather(xg, dst, src)
        hh = _edge_mlp(gi, gj, wa_pad, _bn_scale(p['ga']), p['ba'][None, :],
                       p['Wb'], _bn_scale(p['gb']), p['bb'][None, :])
        sk = _skip_mm(h, p['Ws'], _bn_scale(p['gs']), p['bs'][None, :])
        h = _scatter_max(hh, dst3, sk)

    npad = 10112
    ypad = jnp.pad(h, ((0, npad - N_NODES), (0, 0)))
    bpad = jnp.pad(batch, (0, npad - N_NODES), constant_values=N_GRAPHS)
    bcols = bpad.reshape(1, npad)
    brows = bpad.reshape(npad, 1)

    c = params['cls']
    return _pool_classify(
        ypad, bcols, brows,
        c['W1'], c['b1'][None, :], _bn_scale(c['g1']), c['be1'][None, :],
        c['W2'], c['b2'][None, :], _bn_scale(c['g2']), c['be2'][None, :],
        c['W3'], c['b3'][None, :])


# speedup vs baseline: 1.5118x; 1.1100x over previous
import functools

import jax
import jax.numpy as jnp
from jax import lax
from jax.experimental import pallas as pl
from jax.experimental.pallas import tpu as pltpu
from jax.experimental.pallas import tpu_sc as plsc

N_NODES = 10000
N_EDGES = 320000
N_GRAPHS = 64

_NC = 2
_NS = 16
_NW = _NC * _NS


def _dot_bf16(a, b):
    return jnp.dot(a.astype(jnp.bfloat16), b.astype(jnp.bfloat16),
                   preferred_element_type=jnp.float32)


def _leaky(x):
    return jnp.where(x > 0, x, 0.2 * x)


def _edge_gather(xpad, dst, src):
    n, c = xpad.shape
    e = dst.shape[0]
    epw = e // _NW
    k = 80
    nch = epw // k
    mesh = plsc.VectorSubcoreMesh(core_axis_name="c", subcore_axis_name="s")
    out = jax.ShapeDtypeStruct((e, c), jnp.float32)

    @functools.partial(
        pl.kernel, mesh=mesh,
        out_type=(out, out),
        scratch_types=[
            pltpu.VMEM((k,), jnp.int32),
            pltpu.VMEM((k,), jnp.int32),
            pltpu.VMEM((k, c), jnp.float32),
            pltpu.VMEM((k, c), jnp.float32),
            pltpu.SemaphoreType.DMA,
            pltpu.SemaphoreType.DMA,
        ],
    )
    def kern(x_hbm, dst_hbm, src_hbm, gi_hbm, gj_hbm, dbuf, sbuf, pbuf, qbuf,
             sem1, sem2):
        wid = lax.axis_index("s") * _NC + lax.axis_index("c")

        def chunk(j, carry):
            base = wid * epw + j * k
            pltpu.sync_copy(dst_hbm.at[pl.ds(base, k)], dbuf)
            pltpu.sync_copy(src_hbm.at[pl.ds(base, k)], sbuf)
            cp1 = pltpu.async_copy(x_hbm.at[dbuf], pbuf, sem1)
            cp2 = pltpu.async_copy(x_hbm.at[sbuf], qbuf, sem2)
            cp1.wait()
            cp2.wait()
            pltpu.sync_copy(pbuf, gi_hbm.at[pl.ds(base, k)])
            pltpu.sync_copy(qbuf, gj_hbm.at[pl.ds(base, k)])
            return carry

        lax.fori_loop(0, nch, chunk, 0)

    return kern(xpad, dst, src)


def _edge_mlp(gi, gj, wa, sa, ba, wb, sb, bb):
    e, cpad = gi.shape
    cout = wa.shape[1]
    cout2 = wb.shape[1]
    blk = 1280
    steps = e // blk

    def body(gi_ref, gj_ref, wa_ref, sa_ref, ba_ref, wb_ref, sb_ref, bb_ref,
             h_ref):
        xi = gi_ref[...]
        dj = gj_ref[...] - xi
        m = jnp.concatenate([xi, dj], axis=1)
        y = _dot_bf16(m, wa_ref[...]) * sa_ref[...] + ba_ref[...]
        h1 = _leaky(y)
        y2 = _dot_bf16(h1, wb_ref[...]) * sb_ref[...] + bb_ref[...]
        h_ref[...] = _leaky(y2)

    return pl.pallas_call(
        body,
        grid=(steps,),
        in_specs=[
            pl.BlockSpec((blk, cpad), lambda i: (i, 0)),
            pl.BlockSpec((blk, cpad), lambda i: (i, 0)),
            pl.BlockSpec((2 * cpad, cout), lambda i: (0, 0)),
            pl.BlockSpec((1, cout), lambda i: (0, 0)),
            pl.BlockSpec((1, cout), lambda i: (0, 0)),
            pl.BlockSpec((cout, cout2), lambda i: (0, 0)),
            pl.BlockSpec((1, cout2), lambda i: (0, 0)),
            pl.BlockSpec((1, cout2), lambda i: (0, 0)),
        ],
        out_specs=pl.BlockSpec((blk, cout2), lambda i: (i, 0)),
        out_shape=jax.ShapeDtypeStruct((e, cout2), jnp.float32),
    )(gi, gj, wa, sa, ba, wb, sb, bb)


def _skip_mm(xin, ws, ss, bs):
    n, cin = xin.shape
    c = ws.shape[1]
    blk = 2000

    def body(x_ref, w_ref, s_ref, b_ref, sk_ref):
        sk_ref[...] = _dot_bf16(x_ref[...], w_ref[...]) * s_ref[...] + b_ref[...]

    return pl.pallas_call(
        body,
        grid=(n // blk,),
        in_specs=[
            pl.BlockSpec((blk, cin), lambda i: (i, 0)),
            pl.BlockSpec((cin, c), lambda i: (0, 0)),
            pl.BlockSpec((1, c), lambda i: (0, 0)),
            pl.BlockSpec((1, c), lambda i: (0, 0)),
        ],
        out_specs=pl.BlockSpec((blk, c), lambda i: (i, 0)),
        out_shape=jax.ShapeDtypeStruct((n, c), jnp.float32),
    )(xin, ws, ss, bs)


def _scatter_max(h, dst3, sk):
    e, c = h.shape
    n = sk.shape[0]
    ch = 512
    steps = e // ch
    nt = 4 if c > 128 else 8

    def body(h_ref, d_ref, out_ref, agg_ref):
        i = pl.program_id(0)

        @pl.when(i == 0)
        def _():
            agg_ref[...] = jnp.full((nt, n, c), -jnp.inf, jnp.float32)

        def upd(g, cc):
            for t in range(nt):
                ee = g * nt + t
                dd = d_ref[0, 0, ee]
                agg_ref[t, pl.ds(dd, 1), :] = jnp.maximum(
                    agg_ref[t, pl.ds(dd, 1), :], h_ref[pl.ds(ee, 1), :])
            return cc

        lax.fori_loop(0, ch // nt, upd, 0)

        @pl.when(i == steps - 1)
        def _():
            a = jnp.max(agg_ref[...], axis=0)
            out_ref[...] = jnp.where(a == -jnp.inf, 0.0, a)

    agg = pl.pallas_call(
        body,
        grid=(steps,),
        in_specs=[
            pl.BlockSpec((ch, c), lambda i: (i, 0)),
            pl.BlockSpec((1, 1, ch), lambda i: (i, 0, 0),
                         memory_space=pltpu.SMEM),
        ],
        out_specs=pl.BlockSpec((n, c), lambda i: (0, 0)),
        out_shape=jax.ShapeDtypeStruct((n, c), jnp.float32),
        scratch_shapes=[pltpu.VMEM((nt, n, c), jnp.float32)],
    )(h, dst3)

    blk = 2000

    def ebody(a_ref, sk_ref, o_ref):
        o_ref[...] = _leaky(a_ref[...] + sk_ref[...])

    return pl.pallas_call(
        ebody,
        grid=(n // blk,),
        in_specs=[
            pl.BlockSpec((blk, c), lambda i: (i, 0)),
            pl.BlockSpec((blk, c), lambda i: (i, 0)),
        ],
        out_specs=pl.BlockSpec((blk, c), lambda i: (i, 0)),
        out_shape=jax.ShapeDtypeStruct((n, c), jnp.float32),
    )(agg, sk)


def _pool_classify(y, bcols, brows, w1, b1, s1, be1, w2, b2, s2, be2, w3, b3):
    npad, c = y.shape

    def body(y_ref, bc_ref, br_ref, w1_ref, b1_ref, s1_ref, be1_ref, w2_ref,
             b2_ref, s2_ref, be2_ref, w3_ref, b3_ref, out_ref, gmax_ref):
        gids = lax.broadcasted_iota(jnp.int32, (N_GRAPHS, 1), 0)
        onehot = (bc_ref[...] == gids).astype(jnp.float32)
        yv = y_ref[...]
        sums = jnp.dot(onehot, yv, precision=lax.Precision.HIGHEST,
                       preferred_element_type=jnp.float32)
        counts = jnp.sum(onehot, axis=1, keepdims=True)
        gmean = sums / jnp.maximum(counts, 1.0)

        br = br_ref[...]

        def gmax_step(g, cc):
            m = br == g
            ym = jnp.where(m, yv, -jnp.inf)
            gmax_ref[pl.ds(g, 1), :] = jnp.max(ym, axis=0, keepdims=True)
            return cc

        lax.fori_loop(0, N_GRAPHS, gmax_step, 0)
        gmax = gmax_ref[...]
        gmax = jnp.where(gmax == -jnp.inf, 0.0, gmax)

        z = jnp.concatenate([gmean, gmax], axis=1)
        z = _leaky((_dot_bf16(z, w1_ref[...]) + b1_ref[...]) * s1_ref[...]
                   + be1_ref[...])
        z = _leaky((_dot_bf16(z, w2_ref[...]) + b2_ref[...]) * s2_ref[...]
                   + be2_ref[...])
        out_ref[...] = _dot_bf16(z, w3_ref[...]) + b3_ref[...]

    return pl.pallas_call(
        body,
        out_shape=jax.ShapeDtypeStruct((N_GRAPHS, 2), jnp.float32),
        scratch_shapes=[pltpu.VMEM((N_GRAPHS, c), jnp.float32)],
    )(y, bcols, brows, w1, b1, s1, be1, w2, b2, s2, be2, w3, b3)


def _bn_scale(g, eps=1e-5):
    return (g / jnp.sqrt(1.0 + eps))[None, :]


def kernel(x, edge_index, batch, params):
    dst = edge_index[1]
    src = edge_index[0]
    dst3 = dst.reshape(N_EDGES // 512, 1, 512)

    h = x
    for name in ('ec1', 'ec2', 'ec3'):
        p = params[name]
        cin = h.shape[1]
        xg = jnp.pad(h, ((0, 0), (0, 128 - cin))) if cin < 128 else h
        cpin = xg.shape[1]
        wa = p['Wa']
        wa_pad = jnp.zeros((2 * cpin, wa.shape[1]), jnp.float32)
        wa_pad = wa_pad.at[:cin].set(wa[:cin]).at[cpin:cpin + cin].set(wa[cin:])
        gi, gj = _edge_gather(xg, dst, src)
        hh = _edge_mlp(gi, gj, wa_pad, _bn_scale(p['ga']), p['ba'][None, :],
                       p['Wb'], _bn_scale(p['gb']), p['bb'][None, :])
        sk = _skip_mm(h, p['Ws'], _bn_scale(p['gs']), p['bs'][None, :])
        h = _scatter_max(hh, dst3, sk)

    npad = 10112
    ypad = jnp.pad(h, ((0, npad - N_NODES), (0, 0)))
    bpad = jnp.pad(batch, (0, npad - N_NODES), constant_values=N_GRAPHS)
    bcols = bpad.reshape(1, npad)
    brows = bpad.reshape(npad, 1)

    c = params['cls']
    return _pool_classify(
        ypad, bcols, brows,
        c['W1'], c['b1'][None, :], _bn_scale(c['g1']), c['be1'][None, :],
        c['W2'], c['b2'][None, :], _bn_scale(c['g2']), c['be2'][None, :],
        c['W3'], c['b3'][None, :])


# speedup vs baseline: 1.6245x; 1.0745x over previous
import functools

import jax
import jax.numpy as jnp
from jax import lax
from jax.experimental import pallas as pl
from jax.experimental.pallas import tpu as pltpu
from jax.experimental.pallas import tpu_sc as plsc

N_NODES = 10000
N_EDGES = 320000
N_GRAPHS = 64

_NC = 2
_NS = 16
_NW = _NC * _NS


def _dot_bf16(a, b):
    return jnp.dot(a.astype(jnp.bfloat16), b.astype(jnp.bfloat16),
                   preferred_element_type=jnp.float32)


def _leaky(x):
    return jnp.where(x > 0, x, 0.2 * x)


def _edge_gather(xpad, dst, src):
    n, c = xpad.shape
    e = dst.shape[0]
    epw = e // _NW
    k = 80
    nch = epw // k
    mesh = plsc.VectorSubcoreMesh(core_axis_name="c", subcore_axis_name="s")
    out = jax.ShapeDtypeStruct((e, c), jnp.float32)

    @functools.partial(
        pl.kernel, mesh=mesh,
        out_type=(out, out),
        scratch_types=[
            pltpu.VMEM((k,), jnp.int32),
            pltpu.VMEM((k,), jnp.int32),
            pltpu.VMEM((k,), jnp.int32),
            pltpu.VMEM((k,), jnp.int32),
            pltpu.VMEM((k, c), jnp.float32),
            pltpu.VMEM((k, c), jnp.float32),
            pltpu.VMEM((k, c), jnp.float32),
            pltpu.VMEM((k, c), jnp.float32),
            pltpu.SemaphoreType.DMA,
            pltpu.SemaphoreType.DMA,
            pltpu.SemaphoreType.DMA,
            pltpu.SemaphoreType.DMA,
        ],
    )
    def kern(x_hbm, dst_hbm, src_hbm, gi_hbm, gj_hbm, d0, s0, d1, s1,
             p0, q0, p1, q1, sp0, sq0, sp1, sq1):
        wid = lax.axis_index("s") * _NC + lax.axis_index("c")
        base0 = wid * epw
        bufs = ((d0, s0, p0, q0, sp0, sq0), (d1, s1, p1, q1, sp1, sq1))

        def start(j, b):
            d, s, p, q, semp, semq = bufs[b]
            base = base0 + j * k
            pltpu.sync_copy(dst_hbm.at[pl.ds(base, k)], d)
            pltpu.sync_copy(src_hbm.at[pl.ds(base, k)], s)
            pltpu.async_copy(x_hbm.at[d], p, semp)
            pltpu.async_copy(x_hbm.at[s], q, semq)

        def finish(j, b):
            d, s, p, q, semp, semq = bufs[b]
            base = base0 + j * k
            pltpu.make_async_copy(x_hbm.at[d], p, semp).wait()
            pltpu.make_async_copy(x_hbm.at[s], q, semq).wait()
            pltpu.sync_copy(p, gi_hbm.at[pl.ds(base, k)])
            pltpu.sync_copy(q, gj_hbm.at[pl.ds(base, k)])

        start(0, 0)

        def pair(j2, carry):
            j = j2 * 2

            @pl.when(j + 1 < nch)
            def _():
                start(j + 1, 1)

            finish(j, 0)

            @pl.when(j + 2 < nch)
            def _():
                start(j + 2, 0)

            @pl.when(j + 1 < nch)
            def _():
                finish(j + 1, 1)

            return carry

        lax.fori_loop(0, (nch + 1) // 2, pair, 0)

    return kern(xpad, dst, src)


def _edge_mlp(gi, gj, wa, sa, ba, wb, sb, bb):
    e, cpad = gi.shape
    cout = wa.shape[1]
    cout2 = wb.shape[1]
    blk = 1280
    steps = e // blk

    def body(gi_ref, gj_ref, wa_ref, sa_ref, ba_ref, wb_ref, sb_ref, bb_ref,
             h_ref):
        xi = gi_ref[...]
        dj = gj_ref[...] - xi
        m = jnp.concatenate([xi, dj], axis=1)
        y = _dot_bf16(m, wa_ref[...]) * sa_ref[...] + ba_ref[...]
        h1 = _leaky(y)
        y2 = _dot_bf16(h1, wb_ref[...]) * sb_ref[...] + bb_ref[...]
        h_ref[...] = _leaky(y2)

    return pl.pallas_call(
        body,
        grid=(steps,),
        in_specs=[
            pl.BlockSpec((blk, cpad), lambda i: (i, 0)),
            pl.BlockSpec((blk, cpad), lambda i: (i, 0)),
            pl.BlockSpec((2 * cpad, cout), lambda i: (0, 0)),
            pl.BlockSpec((1, cout), lambda i: (0, 0)),
            pl.BlockSpec((1, cout), lambda i: (0, 0)),
            pl.BlockSpec((cout, cout2), lambda i: (0, 0)),
            pl.BlockSpec((1, cout2), lambda i: (0, 0)),
            pl.BlockSpec((1, cout2), lambda i: (0, 0)),
        ],
        out_specs=pl.BlockSpec((blk, cout2), lambda i: (i, 0)),
        out_shape=jax.ShapeDtypeStruct((e, cout2), jnp.float32),
    )(gi, gj, wa, sa, ba, wb, sb, bb)


def _skip_mm(xin, ws, ss, bs):
    n, cin = xin.shape
    c = ws.shape[1]
    blk = 2000

    def body(x_ref, w_ref, s_ref, b_ref, sk_ref):
        sk_ref[...] = _dot_bf16(x_ref[...], w_ref[...]) * s_ref[...] + b_ref[...]

    return pl.pallas_call(
        body,
        grid=(n // blk,),
        in_specs=[
            pl.BlockSpec((blk, cin), lambda i: (i, 0)),
            pl.BlockSpec((cin, c), lambda i: (0, 0)),
            pl.BlockSpec((1, c), lambda i: (0, 0)),
            pl.BlockSpec((1, c), lambda i: (0, 0)),
        ],
        out_specs=pl.BlockSpec((blk, c), lambda i: (i, 0)),
        out_shape=jax.ShapeDtypeStruct((n, c), jnp.float32),
    )(xin, ws, ss, bs)


def _scatter_max(h, dst3, sk):
    e, c = h.shape
    n = sk.shape[0]
    ch = 512
    steps = e // ch
    nt = 4 if c > 128 else 8

    def body(h_ref, d_ref, out_ref, agg_ref):
        i = pl.program_id(0)

        @pl.when(i == 0)
        def _():
            agg_ref[...] = jnp.full((nt, n, c), -jnp.inf, jnp.float32)

        def upd(g, cc):
            for t in range(nt):
                ee = g * nt + t
                dd = d_ref[0, 0, ee]
                agg_ref[t, pl.ds(dd, 1), :] = jnp.maximum(
                    agg_ref[t, pl.ds(dd, 1), :], h_ref[pl.ds(ee, 1), :])
            return cc

        lax.fori_loop(0, ch // nt, upd, 0)

        @pl.when(i == steps - 1)
        def _():
            a = jnp.max(agg_ref[...], axis=0)
            out_ref[...] = jnp.where(a == -jnp.inf, 0.0, a)

    agg = pl.pallas_call(
        body,
        grid=(steps,),
        in_specs=[
            pl.BlockSpec((ch, c), lambda i: (i, 0)),
            pl.BlockSpec((1, 1, ch), lambda i: (i, 0, 0),
                         memory_space=pltpu.SMEM),
        ],
        out_specs=pl.BlockSpec((n, c), lambda i: (0, 0)),
        out_shape=jax.ShapeDtypeStruct((n, c), jnp.float32),
        scratch_shapes=[pltpu.VMEM((nt, n, c), jnp.float32)],
    )(h, dst3)

    blk = 2000

    def ebody(a_ref, sk_ref, o_ref):
        o_ref[...] = _leaky(a_ref[...] + sk_ref[...])

    return pl.pallas_call(
        ebody,
        grid=(n // blk,),
        in_specs=[
            pl.BlockSpec((blk, c), lambda i: (i, 0)),
            pl.BlockSpec((blk, c), lambda i: (i, 0)),
        ],
        out_specs=pl.BlockSpec((blk, c), lambda i: (i, 0)),
        out_shape=jax.ShapeDtypeStruct((n, c), jnp.float32),
    )(agg, sk)


def _pool_classify(y, bcols, brows, w1, b1, s1, be1, w2, b2, s2, be2, w3, b3):
    npad, c = y.shape

    def body(y_ref, bc_ref, br_ref, w1_ref, b1_ref, s1_ref, be1_ref, w2_ref,
             b2_ref, s2_ref, be2_ref, w3_ref, b3_ref, out_ref, gmax_ref):
        gids = lax.broadcasted_iota(jnp.int32, (N_GRAPHS, 1), 0)
        onehot = (bc_ref[...] == gids).astype(jnp.float32)
        yv = y_ref[...]
        sums = jnp.dot(onehot, yv, precision=lax.Precision.HIGHEST,
                       preferred_element_type=jnp.float32)
        counts = jnp.sum(onehot, axis=1, keepdims=True)
        gmean = sums / jnp.maximum(counts, 1.0)

        br = br_ref[...]

        def gmax_step(g, cc):
            m = br == g
            ym = jnp.where(m, yv, -jnp.inf)
            gmax_ref[pl.ds(g, 1), :] = jnp.max(ym, axis=0, keepdims=True)
            return cc

        lax.fori_loop(0, N_GRAPHS, gmax_step, 0)
        gmax = gmax_ref[...]
        gmax = jnp.where(gmax == -jnp.inf, 0.0, gmax)

        z = jnp.concatenate([gmean, gmax], axis=1)
        z = _leaky((_dot_bf16(z, w1_ref[...]) + b1_ref[...]) * s1_ref[...]
                   + be1_ref[...])
        z = _leaky((_dot_bf16(z, w2_ref[...]) + b2_ref[...]) * s2_ref[...]
                   + be2_ref[...])
        out_ref[...] = _dot_bf16(z, w3_ref[...]) + b3_ref[...]

    return pl.pallas_call(
        body,
        out_shape=jax.ShapeDtypeStruct((N_GRAPHS, 2), jnp.float32),
        scratch_shapes=[pltpu.VMEM((N_GRAPHS, c), jnp.float32)],
    )(y, bcols, brows, w1, b1, s1, be1, w2, b2, s2, be2, w3, b3)


def _bn_scale(g, eps=1e-5):
    return (g / jnp.sqrt(1.0 + eps))[None, :]


def kernel(x, edge_index, batch, params):
    dst = edge_index[1]
    src = edge_index[0]
    dst3 = dst.reshape(N_EDGES // 512, 1, 512)

    h = x
    for name in ('ec1', 'ec2', 'ec3'):
        p = params[name]
        cin = h.shape[1]
        xg = jnp.pad(h, ((0, 0), (0, 128 - cin))) if cin < 128 else h
        cpin = xg.shape[1]
        wa = p['Wa']
        wa_pad = jnp.zeros((2 * cpin, wa.shape[1]), jnp.float32)
        wa_pad = wa_pad.at[:cin].set(wa[:cin]).at[cpin:cpin + cin].set(wa[cin:])
        gi, gj = _edge_gather(xg, dst, src)
        hh = _edge_mlp(gi, gj, wa_pad, _bn_scale(p['ga']), p['ba'][None, :],
                       p['Wb'], _bn_scale(p['gb']), p['bb'][None, :])
        sk = _skip_mm(h, p['Ws'], _bn_scale(p['gs']), p['bs'][None, :])
        h = _scatter_max(hh, dst3, sk)

    npad = 10112
    ypad = jnp.pad(h, ((0, npad - N_NODES), (0, 0)))
    bpad = jnp.pad(batch, (0, npad - N_NODES), constant_values=N_GRAPHS)
    bcols = bpad.reshape(1, npad)
    brows = bpad.reshape(npad, 1)

    c = params['cls']
    return _pool_classify(
        ypad, bcols, brows,
        c['W1'], c['b1'][None, :], _bn_scale(c['g1']), c['be1'][None, :],
        c['W2'], c['b2'][None, :], _bn_scale(c['g2']), c['be2'][None, :],
        c['W3'], c['b3'][None, :])
